# baseline (speedup 1.0000x reference)
import jax
import jax.numpy as jnp
from jax.experimental import pallas as pl
from jax.experimental.pallas import tpu as pltpu

_NCHUNK = 8


def _body(x_hbm, a_hbm, c_ref, w1_ref, b1_ref, wbig_ref, bias_ref, o_ref,
          a_vmem, x_vmem, xall, abf, d_vmem, sem_x, sem_a):
    B, N, D = x_hbm.shape
    ch = N // _NCHUNK

    cp_x = pltpu.make_async_copy(x_hbm, x_vmem, sem_x)
    cp_x.start()
    cps_a = []
    for i in range(_NCHUNK):
        cp = pltpu.make_async_copy(
            a_hbm.at[pl.ds(i * ch, ch), :],
            a_vmem.at[pl.ds(i * ch, ch), :],
            sem_a.at[i],
        )
        cp.start()
        cps_a.append(cp)

    cp_x.wait()
    for b in range(B):
        cb = c_ref[b:b + 1, :]
        sc = cb * jax.nn.sigmoid(cb)
        mod = jnp.dot(sc, w1_ref[:, :], preferred_element_type=jnp.float32)
        mod = mod + b1_ref[0:1, :]
        shift = mod[:, :D]
        scale = mod[:, D:]
        xb = x_vmem[b]
        mu = jnp.mean(xb, axis=1, keepdims=True)
        xc = xb - mu
        var = jnp.mean(xc * xc, axis=1, keepdims=True)
        xn = xc * jax.lax.rsqrt(var + 1e-6)
        xm = xn * (1.0 + scale) + shift
        xall[:, D * b:D * (b + 1)] = xm.astype(jnp.bfloat16)

    zall = jnp.dot(xall[:, :], wbig_ref[:, :], preferred_element_type=jnp.float32)
    z0 = zall[:, 0:128]
    z1 = zall[:, 128:256]
    z2 = zall[:, 256:384]

    for i in range(_NCHUNK):
        cps_a[i].wait()
        rows = a_vmem[pl.ds(i * ch, ch), :]
        d_vmem[pl.ds(i * ch, ch), :] = jnp.sum(rows, axis=1, keepdims=True)
        abf[pl.ds(i * ch, ch), :] = rows.astype(jnp.bfloat16)

    s = jax.lax.rsqrt(d_vmem[:, :])
    a = abf[:, :]

    def lap(y):
        u = jnp.dot(a, (s * y).astype(jnp.bfloat16),
                    preferred_element_type=jnp.float32)
        return y - s * u

    t = lap(z2)
    w = lap(z1 + 2.0 * t)
    o_ref[:, :] = z0 - z2 + w + bias_ref[0:1, :]


def kernel(x, adj, c, W1, b1, cheb_w, cheb_b):
    B, N, D = x.shape
    K, _, _, OUT = cheb_w.shape

    c2 = c.reshape(B, D)
    b1r = b1.reshape(1, 2 * D)
    eye = jnp.eye(B, dtype=x.dtype)
    blocks = [
        jnp.pad(jnp.kron(eye, cheb_w[k, 0]), ((0, 0), (0, 128 - B * OUT)))
        for k in range(K)
    ]
    wbig = jnp.concatenate(blocks, axis=1).astype(jnp.bfloat16)
    bias128 = jnp.pad(jnp.tile(cheb_b.reshape(OUT), B), (0, 128 - B * OUT))
    bias128 = bias128.reshape(1, 128)

    out_full = pl.pallas_call(
        _body,
        out_shape=jax.ShapeDtypeStruct((N, 128), jnp.float32),
        in_specs=[
            pl.BlockSpec(memory_space=pl.ANY),
            pl.BlockSpec(memory_space=pl.ANY),
            pl.BlockSpec(memory_space=pltpu.VMEM),
            pl.BlockSpec(memory_space=pltpu.VMEM),
            pl.BlockSpec(memory_space=pltpu.VMEM),
            pl.BlockSpec(memory_space=pltpu.VMEM),
            pl.BlockSpec(memory_space=pltpu.VMEM),
        ],
        scratch_shapes=[
            pltpu.VMEM((N, N), jnp.float32),
            pltpu.VMEM((B, N, D), jnp.float32),
            pltpu.VMEM((N, B * D), jnp.bfloat16),
            pltpu.VMEM((N, N), jnp.bfloat16),
            pltpu.VMEM((N, 1), jnp.float32),
            pltpu.SemaphoreType.DMA,
            pltpu.SemaphoreType.DMA((_NCHUNK,)),
        ],
        compiler_params=pltpu.CompilerParams(
            vmem_limit_bytes=100 * 1024 * 1024,
        ),
    )(x, adj, c2, W1, b1r, wbig, bias128)

    out = out_full[:, :B * OUT].reshape(N, B, OUT).transpose(1, 0, 2)
    return out
